# Initial kernel scaffold; baseline (speedup 1.0000x reference)
import jax
import jax.numpy as jnp
from jax.experimental import pallas as pl


def kernel(input, params):
    raise NotImplementedError("write your pallas kernel here")


# speedup vs baseline: 1.0483x; 1.0483x over previous
import functools

import jax
import jax.numpy as jnp
from jax.experimental import pallas as pl
from jax.experimental.pallas import tpu as pltpu


def _conv(x, w, b, stride=1, pad=1):
    o = jax.lax.conv_general_dilated(
        x, w, (stride, stride), [(pad, pad), (pad, pad)],
        dimension_numbers=('NCHW', 'OIHW', 'NCHW'))
    return o + b[None, :, None, None]


def _convT(x, w, b, stride=2, pad=1):
    k = w.shape[2]
    p = k - 1 - pad
    o = jax.lax.conv_general_dilated(
        x, jnp.flip(w, (2, 3)), (1, 1), [(p, p), (p, p)],
        lhs_dilation=(stride, stride), dimension_numbers=('NCHW', 'OIHW', 'NCHW'))
    return o + b[None, :, None, None]


def _res(x, p, pre):
    h = jax.nn.relu(x)
    h = _conv(h, p[pre + '_w1'], p[pre + '_b1'], 1, 1)
    h = jax.nn.relu(h)
    h = _conv(h, p[pre + '_w2'], p[pre + '_b2'], 1, 0)
    return x + h


def _vq_body(flat_ref, fsq_ref, emb_ref, embT_ref, e2_ref, q_ref, dsum_ref):
    flat = flat_ref[...]
    scores = jax.lax.dot(flat.astype(jnp.bfloat16),
                         emb_ref[...].astype(jnp.bfloat16),
                         preferred_element_type=jnp.float32)
    dist = fsq_ref[...] - 2.0 * scores + e2_ref[...]
    m = jnp.min(dist, axis=1, keepdims=True)
    C = dist.shape[1]
    iota = jax.lax.broadcasted_iota(jnp.int32, dist.shape, 1)
    ind = jnp.min(jnp.where(dist == m, iota, C), axis=1)
    onehot = (iota == ind[:, None]).astype(jnp.float32)
    q = jax.lax.dot(onehot, embT_ref[...],
                    precision=jax.lax.Precision.HIGHEST)
    q_ref[...] = q
    d = q - flat
    dsum = jnp.sum(d * d) * jnp.ones((1, 1), jnp.float32)

    @pl.when(pl.program_id(0) == 0)
    def _init():
        dsum_ref[...] = jnp.zeros((1, 1), jnp.float32)

    dsum_ref[...] += dsum


def _vq(flat, embed, block_rows):
    N, D = flat.shape
    C = embed.shape[1]
    fsq = (flat ** 2).sum(1, keepdims=True)
    e2 = (embed ** 2).sum(0, keepdims=True)
    grid = N // block_rows
    q, dsum = pl.pallas_call(
        _vq_body,
        grid=(grid,),
        in_specs=[
            pl.BlockSpec((block_rows, D), lambda i: (i, 0)),
            pl.BlockSpec((block_rows, 1), lambda i: (i, 0)),
            pl.BlockSpec((D, C), lambda i: (0, 0)),
            pl.BlockSpec((C, D), lambda i: (0, 0)),
            pl.BlockSpec((1, C), lambda i: (0, 0)),
        ],
        out_specs=[
            pl.BlockSpec((block_rows, D), lambda i: (i, 0)),
            pl.BlockSpec((1, 1), lambda i: (0, 0)),
        ],
        out_shape=[
            jax.ShapeDtypeStruct((N, D), jnp.float32),
            jax.ShapeDtypeStruct((1, 1), jnp.float32),
        ],
    )(flat, fsq, embed, embed.T, e2)
    return q, dsum[0, 0]


def _quantize_nchw(x_nchw, w1x1, b1x1, embed, block_rows):
    flat_nhwc = _conv(x_nchw, w1x1, b1x1, 1, 0).transpose(0, 2, 3, 1)
    Bn, H, Wd, D = flat_nhwc.shape
    flat = flat_nhwc.reshape(-1, D)
    q, dsum = _vq(flat, embed, block_rows)
    diff = dsum / (flat.shape[0] * D)
    quant = q.reshape(Bn, H, Wd, D).transpose(0, 3, 1, 2)
    return quant, diff


def kernel(input, params):
    p = params
    h = _conv(input, p['eb_w1'], p['eb_b1'], 2, 1); h = jax.nn.relu(h)
    h = _conv(h, p['eb_w2'], p['eb_b2'], 2, 1); h = jax.nn.relu(h)
    h = _conv(h, p['eb_w3'], p['eb_b3'], 1, 1)
    h = _res(h, p, 'eb_r1'); h = _res(h, p, 'eb_r2')
    enc_b = jax.nn.relu(h)
    h = _conv(enc_b, p['et_w1'], p['et_b1'], 2, 1); h = jax.nn.relu(h)
    h = _conv(h, p['et_w2'], p['et_b2'], 1, 1)
    h = _res(h, p, 'et_r1'); h = _res(h, p, 'et_r2')
    enc_t = jax.nn.relu(h)

    quant_t, diff_t = _quantize_nchw(enc_t, p['qct_w'], p['qct_b'],
                                     p['embed_t'], block_rows=448)

    h = _conv(quant_t, p['dt_w1'], p['dt_b1'], 1, 1)
    h = _res(h, p, 'dt_r1'); h = _res(h, p, 'dt_r2')
    h = jax.nn.relu(h)
    dec_t = _convT(h, p['dt_wt'], p['dt_bt'], 2, 1)
    cat_b = jnp.concatenate([dec_t, enc_b], axis=1)

    quant_b, diff_b = _quantize_nchw(cat_b, p['qcb_w'], p['qcb_b'],
                                     p['embed_b'], block_rows=448)

    up_t = _convT(quant_t, p['up_wt'], p['up_bt'], 2, 1)
    quant = jnp.concatenate([up_t, quant_b], axis=1)
    h = _conv(quant, p['d_w1'], p['d_b1'], 1, 1)
    h = _res(h, p, 'd_r1'); h = _res(h, p, 'd_r2')
    h = jax.nn.relu(h)
    h = _convT(h, p['d_wt1'], p['d_bt1'], 2, 1); h = jax.nn.relu(h)
    dec = _convT(h, p['d_wt2'], p['d_bt2'], 2, 1)
    diff = diff_t[None] + diff_b[None]
    return dec, diff
